# Initial kernel scaffold; baseline (speedup 1.0000x reference)
import jax
import jax.numpy as jnp
from jax.experimental import pallas as pl


def kernel(X, A, W1, b1, W2, b2, W3, b3):
    raise NotImplementedError("write your pallas kernel here")


# speedup vs baseline: 1.4242x; 1.4242x over previous
import jax
import jax.numpy as jnp
from jax.experimental import pallas as pl


def _cheb_layer(L, x, W, b):
    t1 = jnp.dot(L, x, preferred_element_type=jnp.float32)
    t2 = 2.0 * jnp.dot(L, t1, preferred_element_type=jnp.float32) - x
    out = jnp.dot(x, W[0], preferred_element_type=jnp.float32)
    out = out + jnp.dot(t1, W[1], preferred_element_type=jnp.float32)
    out = out + jnp.dot(t2, W[2], preferred_element_type=jnp.float32)
    return out + b


def _net_kernel(x_ref, a_ref, w1_ref, b1_ref, w2_ref, b2_ref, w3_ref, b3_ref,
                o_ref):
    A = a_ref[0]
    deg = jnp.sum(A, axis=-1)
    dinv = jnp.where(deg > 0.0, jax.lax.rsqrt(jnp.where(deg > 0.0, deg, 1.0)),
                     0.0)
    L = -(A * dinv[:, None] * dinv[None, :])

    x = x_ref[0]
    h = jax.nn.relu(_cheb_layer(L, x, w1_ref[...], b1_ref[...]))
    h = jax.nn.relu(_cheb_layer(L, h, w2_ref[...], b2_ref[...]))
    o_ref[0] = _cheb_layer(L, h, w3_ref[...], b3_ref[...])


def kernel(X, A, W1, b1, W2, b2, W3, b3):
    B, S, T, E = X.shape
    d_in = T * E
    d_hid = W1.shape[-1]
    d_out = W3.shape[-1]
    x = X.reshape(B, S, d_in)

    def batch_spec(shape):
        return pl.BlockSpec((1,) + shape, lambda b: (b, 0, 0))

    def full_spec(arr):
        return pl.BlockSpec(arr.shape, lambda b: (0,) * arr.ndim)

    return pl.pallas_call(
        _net_kernel,
        grid=(B,),
        in_specs=[
            batch_spec((S, d_in)),
            batch_spec((S, S)),
            full_spec(W1), full_spec(b1),
            full_spec(W2), full_spec(b2),
            full_spec(W3), full_spec(b3),
        ],
        out_specs=batch_spec((S, d_out)),
        out_shape=jax.ShapeDtypeStruct((B, S, d_out), jnp.float32),
    )(x, A, W1, b1, W2, b2, W3, b3)
